# Initial kernel scaffold; baseline (speedup 1.0000x reference)
import jax
import jax.numpy as jnp
from jax.experimental import pallas as pl


def kernel(x_cat, tables):
    raise NotImplementedError("write your pallas kernel here")


# speedup vs baseline: 4.3088x; 4.3088x over previous
import functools

import jax
import jax.numpy as jnp
from jax import lax
from jax.experimental import pallas as pl
from jax.experimental.pallas import tpu as pltpu
from jax.experimental.pallas import tpu_sc as plsc

_NW = 32
_CHUNK = 128
_LANES = 16


@functools.lru_cache(maxsize=None)
def _build(rows_total, n_rows_tab, emb):
    rows_per_w = rows_total // _NW
    n_chunks = rows_per_w // _CHUNK
    n_pairs = n_chunks // 2
    vec_per_chunk = _CHUNK // _LANES

    mesh = plsc.VectorSubcoreMesh(core_axis_name="c", subcore_axis_name="s")

    @functools.partial(
        pl.kernel,
        mesh=mesh,
        compiler_params=pltpu.CompilerParams(use_tc_tiling_on_sc=False),
        out_type=jax.ShapeDtypeStruct((rows_total, emb), jnp.float32),
        scratch_types=[
            pltpu.VMEM((rows_per_w,), jnp.int32),
            pltpu.VMEM((rows_per_w,), jnp.int32),
            pltpu.VMEM((n_chunks, _CHUNK), jnp.int32),
            pltpu.VMEM((_CHUNK, emb), jnp.float32),
            pltpu.VMEM((_CHUNK, emb), jnp.float32),
            pltpu.SemaphoreType.DMA,
            pltpu.SemaphoreType.DMA,
            pltpu.SemaphoreType.DMA,
            pltpu.SemaphoreType.DMA,
        ],
    )
    def k(xflat, offs, tab, out, xc_v, offs_v, idx_v, buf0, buf1, g0, g1, s0, s1):
        cid = lax.axis_index("c")
        sid = lax.axis_index("s")
        wid = sid * 2 + cid
        base = wid * rows_per_w

        pltpu.sync_copy(xflat.at[pl.ds(base, rows_per_w)], xc_v)
        pltpu.sync_copy(offs, offs_v)

        def idx_body(j, _):
            for l in range(vec_per_chunk):
                fo = j * _CHUNK + l * _LANES
                idx_v[j, pl.ds(l * _LANES, _LANES)] = (
                    xc_v[pl.ds(fo, _LANES)] + offs_v[pl.ds(fo, _LANES)]
                )
            return 0

        lax.fori_loop(0, n_chunks, idx_body, 0)

        bufs = (buf0, buf1)
        gsems = (g0, g1)
        ssems = (s0, s1)

        def gather(j, b):
            return pltpu.make_async_copy(tab.at[idx_v.at[j]], bufs[b], gsems[b])

        def store(j, b):
            return pltpu.make_async_copy(
                bufs[b], out.at[pl.ds(base + j * _CHUNK, _CHUNK)], ssems[b]
            )

        gather(0, 0).start()
        gather(1, 1).start()

        def pair_body(g, _):
            for b in range(2):
                j = g * 2 + b
                gather(j, b).wait()
                store(j, b).start()

                @pl.when(g < n_pairs - 1)
                def _():
                    store(j, b).wait()
                    gather(j + 2, b).start()

            return 0

        lax.fori_loop(0, n_pairs, pair_body, 0)

        store(n_chunks - 2, 0).wait()
        store(n_chunks - 1, 1).wait()

    return k


def kernel(x_cat, tables):
    batch, nf = x_cat.shape
    nf2, vocab, emb = tables.shape
    rows_total = batch * nf
    rows_per_w = rows_total // _NW

    xflat = x_cat.reshape(rows_total).astype(jnp.int32)
    tab = tables.reshape(nf2 * vocab, emb)
    offs = jnp.tile(jnp.arange(nf, dtype=jnp.int32) * vocab, rows_per_w // nf)

    out = _build(rows_total, nf2 * vocab, emb)(xflat, offs, tab)
    return out.reshape(batch, nf, emb)


# speedup vs baseline: 5.0673x; 1.1760x over previous
import functools

import jax
import jax.numpy as jnp
from jax import lax
from jax.experimental import pallas as pl
from jax.experimental.pallas import tpu as pltpu
from jax.experimental.pallas import tpu_sc as plsc

_NW = 32
_CHUNK = 128
_LANES = 16


@functools.lru_cache(maxsize=None)
def _build(rows_total, n_rows_tab, emb):
    rows_per_w = rows_total // _NW
    n_chunks = rows_per_w // _CHUNK
    n_pairs = n_chunks // 2
    vec_per_chunk = _CHUNK // _LANES

    mesh = plsc.VectorSubcoreMesh(core_axis_name="c", subcore_axis_name="s")

    @functools.partial(
        pl.kernel,
        mesh=mesh,
        compiler_params=pltpu.CompilerParams(use_tc_tiling_on_sc=False),
        out_type=jax.ShapeDtypeStruct((rows_total, emb), jnp.float32),
        scratch_types=[
            pltpu.VMEM((rows_per_w,), jnp.int32),
            pltpu.VMEM((rows_per_w,), jnp.int32),
            pltpu.VMEM((n_chunks, _CHUNK), jnp.int32),
            pltpu.VMEM((_CHUNK, emb), jnp.float32),
            pltpu.VMEM((_CHUNK, emb), jnp.float32),
            pltpu.VMEM_SHARED((n_rows_tab, emb), jnp.float32),
            pltpu.SemaphoreType.DMA,
            pltpu.SemaphoreType.DMA,
            pltpu.SemaphoreType.DMA,
            pltpu.SemaphoreType.DMA,
        ],
    )
    def k(xflat, offs, tab, out, xc_v, offs_v, idx_v, buf0, buf1, tab_sh,
          g0, g1, s0, s1):
        cid = lax.axis_index("c")
        sid = lax.axis_index("s")
        wid = sid * 2 + cid
        base = wid * rows_per_w

        @pl.when(sid == 0)
        def _():
            pltpu.sync_copy(tab, tab_sh)

        pltpu.sync_copy(xflat.at[pl.ds(base, rows_per_w)], xc_v)
        pltpu.sync_copy(offs, offs_v)

        def idx_body(j, _):
            for l in range(vec_per_chunk):
                fo = j * _CHUNK + l * _LANES
                idx_v[j, pl.ds(l * _LANES, _LANES)] = (
                    xc_v[pl.ds(fo, _LANES)] + offs_v[pl.ds(fo, _LANES)]
                )
            return 0

        lax.fori_loop(0, n_chunks, idx_body, 0)

        bufs = (buf0, buf1)
        gsems = (g0, g1)
        ssems = (s0, s1)

        plsc.subcore_barrier()

        def gather(j, b):
            return pltpu.make_async_copy(
                tab_sh.at[idx_v.at[j]], bufs[b], gsems[b]
            )

        def store(j, b):
            return pltpu.make_async_copy(
                bufs[b], out.at[pl.ds(base + j * _CHUNK, _CHUNK)], ssems[b]
            )

        gather(0, 0).start()
        gather(1, 1).start()

        def pair_body(g, _):
            for b in range(2):
                j = g * 2 + b
                gather(j, b).wait()
                store(j, b).start()

                @pl.when(g < n_pairs - 1)
                def _():
                    store(j, b).wait()
                    gather(j + 2, b).start()

            return 0

        lax.fori_loop(0, n_pairs, pair_body, 0)

        store(n_chunks - 2, 0).wait()
        store(n_chunks - 1, 1).wait()

    return k


def kernel(x_cat, tables):
    batch, nf = x_cat.shape
    nf2, vocab, emb = tables.shape
    rows_total = batch * nf
    rows_per_w = rows_total // _NW

    xflat = x_cat.reshape(rows_total).astype(jnp.int32)
    tab = tables.reshape(nf2 * vocab, emb)
    offs = jnp.tile(jnp.arange(nf, dtype=jnp.int32) * vocab, rows_per_w // nf)

    out = _build(rows_total, nf2 * vocab, emb)(xflat, offs, tab)
    return out.reshape(batch, nf, emb)


# speedup vs baseline: 5.0779x; 1.0021x over previous
import functools

import jax
import jax.numpy as jnp
from jax import lax
from jax.experimental import pallas as pl
from jax.experimental.pallas import tpu as pltpu
from jax.experimental.pallas import tpu_sc as plsc

_NW = 32
_CHUNK = 128
_K = 4
_LANES = 16


@functools.lru_cache(maxsize=None)
def _build(rows_total, n_rows_tab, emb):
    rows_per_w = rows_total // _NW
    gather_rows = _K * _CHUNK
    n_gathers = rows_per_w // gather_rows
    n_pairs = n_gathers // 2
    vec_per_gather = gather_rows // _LANES

    mesh = plsc.VectorSubcoreMesh(core_axis_name="c", subcore_axis_name="s")

    @functools.partial(
        pl.kernel,
        mesh=mesh,
        compiler_params=pltpu.CompilerParams(use_tc_tiling_on_sc=False),
        out_type=jax.ShapeDtypeStruct((rows_total, emb), jnp.float32),
        scratch_types=[
            pltpu.VMEM((rows_per_w,), jnp.int32),
            pltpu.VMEM((rows_per_w,), jnp.int32),
            pltpu.VMEM((n_gathers, gather_rows), jnp.int32),
            pltpu.VMEM((gather_rows, emb), jnp.float32),
            pltpu.VMEM((gather_rows, emb), jnp.float32),
            pltpu.VMEM_SHARED((n_rows_tab, emb), jnp.float32),
            pltpu.SemaphoreType.DMA,
            pltpu.SemaphoreType.DMA,
            pltpu.SemaphoreType.DMA,
            pltpu.SemaphoreType.DMA,
        ],
    )
    def k(xflat, offs, tab, out, xc_v, offs_v, idx_v, buf0, buf1, tab_sh,
          g0, g1, s0, s1):
        cid = lax.axis_index("c")
        sid = lax.axis_index("s")
        wid = sid * 2 + cid
        base = wid * rows_per_w

        @pl.when(sid == 0)
        def _():
            pltpu.sync_copy(tab, tab_sh)

        pltpu.sync_copy(xflat.at[pl.ds(base, rows_per_w)], xc_v)
        pltpu.sync_copy(offs, offs_v)

        def idx_body(j, _):
            for l in range(vec_per_gather):
                fo = j * gather_rows + l * _LANES
                idx_v[j, pl.ds(l * _LANES, _LANES)] = (
                    xc_v[pl.ds(fo, _LANES)] + offs_v[pl.ds(fo, _LANES)]
                )
            return 0

        lax.fori_loop(0, n_gathers, idx_body, 0)

        bufs = (buf0, buf1)
        gsems = (g0, g1)
        ssems = (s0, s1)

        plsc.subcore_barrier()

        def gather(j, b):
            return pltpu.make_async_copy(
                tab_sh.at[idx_v.at[j]], bufs[b], gsems[b]
            )

        def store(j, b):
            return pltpu.make_async_copy(
                bufs[b],
                out.at[pl.ds(base + j * gather_rows, gather_rows)],
                ssems[b],
            )

        gather(0, 0).start()
        gather(1, 1).start()

        def pair_body(g, _):
            for b in range(2):
                j = g * 2 + b
                gather(j, b).wait()
                store(j, b).start()

                @pl.when(g < n_pairs - 1)
                def _():
                    store(j, b).wait()
                    gather(j + 2, b).start()

            return 0

        lax.fori_loop(0, n_pairs, pair_body, 0)

        store(n_gathers - 2, 0).wait()
        store(n_gathers - 1, 1).wait()

    return k


def kernel(x_cat, tables):
    batch, nf = x_cat.shape
    nf2, vocab, emb = tables.shape
    rows_total = batch * nf
    rows_per_w = rows_total // _NW

    xflat = x_cat.reshape(rows_total).astype(jnp.int32)
    tab = tables.reshape(nf2 * vocab, emb)
    offs = jnp.tile(jnp.arange(nf, dtype=jnp.int32) * vocab, rows_per_w // nf)

    out = _build(rows_total, nf2 * vocab, emb)(xflat, offs, tab)
    return out.reshape(batch, nf, emb)
